# Initial kernel scaffold; baseline (speedup 1.0000x reference)
import jax
import jax.numpy as jnp
from jax.experimental import pallas as pl


def kernel(x, edge_index, W, b):
    raise NotImplementedError("write your pallas kernel here")


# speedup vs baseline: 20.8497x; 20.8497x over previous
import functools

import jax
import jax.numpy as jnp
from jax import lax
from jax.experimental import pallas as pl
from jax.experimental.pallas import tpu as pltpu
from jax.experimental.pallas import tpu_sc as plsc

NC = 2
NS = 16
NW = NC * NS
L = 16
CH = 128
NBUF = 1


def _sc_mesh():
    return plsc.VectorSubcoreMesh(core_axis_name="c", subcore_axis_name="s",
                                  num_cores=NC, num_subcores=NS)


def _make_deg_kernel(ept, n16):
    @functools.partial(
        pl.kernel,
        out_type=jax.ShapeDtypeStruct((NW, n16), jnp.float32),
        mesh=_sc_mesh(),
        scratch_types=[
            pltpu.VMEM((ept,), jnp.int32),
            pltpu.VMEM((n16,), jnp.float32),
        ],
        compiler_params=pltpu.CompilerParams(needs_layout_passes=False),
    )
    def deg_kernel(dst_hbm, deg_out, dst_v, deg_v):
        c = lax.axis_index("c")
        s = lax.axis_index("s")
        t = c * NS + s
        pltpu.sync_copy(dst_hbm.at[t], dst_v)

        zeros = jnp.zeros((L,), jnp.float32)

        @pl.loop(0, n16 // L)
        def _(i):
            deg_v[pl.ds(i * L, L)] = zeros

        ones = jnp.ones((L,), jnp.float32)

        @pl.loop(0, ept // L)
        def _(i):
            idx = dst_v[pl.ds(i * L, L)]
            plsc.addupdate_scatter(deg_v, [idx], ones)

        pltpu.sync_copy(deg_v, deg_out.at[t])

    return deg_kernel


def _make_scale_kernel(n16, d):
    def body(xp_ref, degp_ref, g_ref, dis_ref):
        deg = 1.0 + jnp.sum(degp_ref[...], axis=0)
        dis = lax.rsqrt(deg)
        dis_ref[...] = dis[:, None]
        g_ref[...] = xp_ref[...] * dis[:, None]

    return pl.pallas_call(
        body,
        out_shape=(
            jax.ShapeDtypeStruct((n16, d), jnp.float32),
            jax.ShapeDtypeStruct((n16, 1), jnp.float32),
        ),
    )


def _make_scatter_kernel(nchunk, n16, n, d):
    zr = n16 // NS

    @functools.partial(
        pl.kernel,
        out_type=jax.ShapeDtypeStruct((NC, n16, d), jnp.float32),
        mesh=_sc_mesh(),
        scratch_types=[
            pltpu.VMEM((nchunk, CH), jnp.int32),
            pltpu.VMEM((nchunk, CH), jnp.int32),
            pltpu.VMEM((CH, d), jnp.float32),
            pltpu.VMEM_SHARED((n16, d), jnp.float32),
            pltpu.SemaphoreType.DMA,
        ],
    )
    def scatter_kernel(g_hbm, src_hbm, dst_hbm, zero_hbm, acc_out,
                       src_v, dst_v, buf0, acc_sh, sem0):
        c = lax.axis_index("c")
        s = lax.axis_index("s")
        t = c * NS + s
        bufs = (buf0,)
        sems = (sem0,)

        pltpu.sync_copy(src_hbm.at[t], src_v)
        pltpu.sync_copy(dst_hbm.at[t], dst_v)

        for b in range(NBUF):
            pltpu.async_copy(g_hbm.at[src_v.at[b]], bufs[b], sems[b])

        pltpu.sync_copy(zero_hbm.at[pl.ds(s * zr, zr)],
                        acc_sh.at[pl.ds(s * zr, zr)])
        plsc.subcore_barrier()

        @pl.loop(0, (nchunk - NBUF) // NBUF)
        def _(i):
            j = i * NBUF
            for b in range(NBUF):
                pltpu.make_async_copy(
                    g_hbm.at[src_v.at[j + b]], bufs[b], sems[b]).wait()
                pltpu.sync_copy(bufs[b], acc_sh.at[dst_v.at[j + b]], add=True)
                pltpu.async_copy(
                    g_hbm.at[src_v.at[j + b + NBUF]], bufs[b], sems[b])

        for b in range(NBUF):
            j = nchunk - NBUF + b
            pltpu.make_async_copy(
                g_hbm.at[src_v.at[j]], bufs[b], sems[b]).wait()
            pltpu.sync_copy(bufs[b], acc_sh.at[dst_v.at[j]], add=True)

        plsc.subcore_barrier()
        pltpu.sync_copy(acc_sh.at[pl.ds(s * zr, zr)],
                        acc_out.at[c, pl.ds(s * zr, zr)])

    return scatter_kernel


def _make_final_kernel(n, n16, d, rb):
    def body(acc_ref, g_ref, dis_ref, w_ref, b_ref, out_ref):
        t = acc_ref[0] + acc_ref[1] + g_ref[...]
        t = t * dis_ref[...]
        out_ref[...] = (
            jnp.dot(t, w_ref[...], preferred_element_type=jnp.float32)
            + b_ref[0, :]
        )

    return pl.pallas_call(
        body,
        grid=(n // rb,),
        in_specs=[
            pl.BlockSpec((NC, rb, d), lambda i: (0, i, 0)),
            pl.BlockSpec((rb, d), lambda i: (i, 0)),
            pl.BlockSpec((rb, 1), lambda i: (i, 0)),
            pl.BlockSpec((d, d), lambda i: (0, 0)),
            pl.BlockSpec((1, d), lambda i: (0, 0)),
        ],
        out_specs=pl.BlockSpec((rb, d), lambda i: (i, 0)),
        out_shape=jax.ShapeDtypeStruct((n, d), jnp.float32),
    )


def kernel(x, edge_index, W, b):
    n, d = x.shape
    e = edge_index.shape[1]

    nchunk = -(-e // (NW * CH))
    if nchunk % NBUF:
        nchunk += NBUF - nchunk % NBUF
    e_pad = NW * nchunk * CH
    n16 = -(-(n + 1) // (NS * 8)) * (NS * 8)

    src = edge_index[0]
    dst = edge_index[1]
    pad = jnp.full((e_pad - e,), n, dtype=jnp.int32)
    src_p = jnp.concatenate([src, pad]).reshape(NW, nchunk, CH)
    dst_p = jnp.concatenate([dst, pad]).reshape(NW, nchunk, CH)

    x_p = jnp.zeros((n16, d), x.dtype).at[:n].set(x)

    deg_parts = _make_deg_kernel(nchunk * CH, n16)(dst_p.reshape(NW, -1))
    g, dis = _make_scale_kernel(n16, d)(x_p, deg_parts)
    zeros = jnp.zeros((n16, d), jnp.float32)
    accs = _make_scatter_kernel(nchunk, n16, n, d)(g, src_p, dst_p, zeros)
    rb = 2000 if n % 2000 == 0 else n
    out = _make_final_kernel(n, n16, d, rb)(accs, g, dis, W, b.reshape(1, d))
    return out
